# baseline (speedup 1.0000x reference)
import functools

import jax
import jax.numpy as jnp
from jax import lax
from jax.experimental import pallas as pl
from jax.experimental.pallas import tpu as pltpu
from jax.experimental.pallas import tpu_sc as plsc

DIM = 64
K = 20
NW = 32
CB = 32
ROWS = CB * K
GATHERS = ROWS // 128


def _sc_word2vec(B, interpret=False):
    b_per_w = B // NW
    n_chunks = b_per_w // CB
    pairs_per_w = b_per_w * K
    ctx_rows_per_w = pairs_per_w // 128

    mesh = plsc.VectorSubcoreMesh(core_axis_name="c", subcore_axis_name="s",
                                  num_cores=2, num_subcores=16)

    @functools.partial(
        pl.kernel,
        out_type=jax.ShapeDtypeStruct((B * K,), jnp.float32),
        mesh=mesh,
        scratch_types=[
            pltpu.VMEM((n_chunks, CB), jnp.int32),
            pltpu.VMEM((ctx_rows_per_w, 128), jnp.int32),
            pltpu.VMEM((CB, DIM), jnp.float32),
            pltpu.VMEM((ROWS, DIM), jnp.float32),
            pltpu.VMEM((ROWS,), jnp.float32),
            pltpu.SemaphoreType.DMA,
            pltpu.SemaphoreType.DMA,
        ],
        compiler_params=pltpu.CompilerParams(use_tc_tiling_on_sc=False,
                                             needs_layout_passes=False),
        interpret=interpret,
    )
    def k(center2d, ctx2d, in_embed, out_embed, out_hbm,
          cidx_v, ctxidx_v, cent_v, rows_v, out_v, sem0, sem1):
        nc = 2
        wid = lax.axis_index("s") * nc + lax.axis_index("c")
        pltpu.sync_copy(center2d.at[pl.ds(wid * n_chunks, n_chunks)], cidx_v)
        pltpu.sync_copy(ctx2d.at[pl.ds(wid * ctx_rows_per_w, ctx_rows_per_w)],
                        ctxidx_v)

        def chunk_body(j, _):
            ccopy = pltpu.async_copy(in_embed.at[cidx_v.at[j]], cent_v, sem0)
            rcopies = [
                pltpu.async_copy(
                    out_embed.at[ctxidx_v.at[j * GATHERS + i]],
                    rows_v.at[pl.ds(i * 128, 128)], sem1)
                for i in range(GATHERS)
            ]
            ccopy.wait()
            for c in rcopies:
                c.wait()

            def group_body(g, _):
                lanes = lax.iota(jnp.int32, 16)
                prow = g * 16 + lanes
                brow = lax.div(prow, jnp.full((16,), K, jnp.int32))

                def d_body(d, t):
                    dv = jnp.full((16,), d, jnp.int32)
                    u = plsc.load_gather(rows_v, [prow, dv])
                    v = plsc.load_gather(cent_v, [brow, dv])
                    return t + u * v

                t = lax.fori_loop(0, DIM, d_body, jnp.zeros((16,), jnp.float32))
                out_v[pl.ds(g * 16, 16)] = t
                return 0

            lax.fori_loop(0, ROWS // 16, group_body, 0)
            pltpu.sync_copy(
                out_v, out_hbm.at[pl.ds(wid * pairs_per_w + j * ROWS, ROWS)])
            return 0

        lax.fori_loop(0, n_chunks, chunk_body, 0)

    return k


def kernel(center_words, context_words, in_embed, out_embed):
    B, Kk = context_words.shape
    assert Kk == K and in_embed.shape[1] == DIM
    center2d = center_words.astype(jnp.int32).reshape(B // CB, CB)
    ctx2d = context_words.astype(jnp.int32).reshape(B * K // 128, 128)
    out_flat = _sc_word2vec(B)(center2d, ctx2d, in_embed, out_embed)
    return out_flat.reshape(B, K)


# speedup vs baseline: 1.0534x; 1.0534x over previous
import functools

import jax
import jax.numpy as jnp
from jax import lax
from jax.experimental import pallas as pl
from jax.experimental.pallas import tpu as pltpu
from jax.experimental.pallas import tpu_sc as plsc

DIM = 64
K = 20
NW = 32
CB = 8
ROWS = CB * K


def _sc_word2vec(B, interpret=False):
    b_per_w = B // NW
    n_chunks = b_per_w // CB
    pairs_per_w = b_per_w * K
    groups = ROWS // 16
    ctx_rows = pairs_per_w // 128
    cent_rows = 8

    mesh = plsc.VectorSubcoreMesh(core_axis_name="c", subcore_axis_name="s",
                                  num_cores=2, num_subcores=16)

    @functools.partial(
        pl.kernel,
        out_type=jax.ShapeDtypeStruct((B * K,), jnp.float32),
        mesh=mesh,
        scratch_types=[
            pltpu.VMEM((ctx_rows, 128), jnp.int32),
            pltpu.VMEM((ctx_rows, 128), jnp.int32),
            pltpu.VMEM((cent_rows, 128), jnp.int32),
            pltpu.VMEM((cent_rows, 128), jnp.int32),
            pltpu.VMEM((CB, 128), jnp.float32),
            pltpu.VMEM((CB, 128), jnp.float32),
            pltpu.VMEM((ROWS, 128), jnp.float32),
            pltpu.VMEM((ROWS, 128), jnp.float32),
            pltpu.VMEM((pairs_per_w,), jnp.float32),
            pltpu.SemaphoreType.DMA,
            pltpu.SemaphoreType.DMA,
        ],
        compiler_params=pltpu.CompilerParams(needs_layout_passes=False),
        interpret=interpret,
    )
    def k(hctx3d, ctx3d, hcent3d, cent3d, in_embed2, out_embed2, out_hbm,
          hctx_v, ctx_v, hcid_v, cid_v, cent_a, cent_b, rows_a, rows_b,
          out_v, gsem0, gsem1):
        nc = 2
        wid = lax.axis_index("s") * nc + lax.axis_index("c")
        cent_bufs = (cent_a, cent_b)
        rows_bufs = (rows_a, rows_b)
        gsems = (gsem0, gsem1)

        pltpu.sync_copy(hctx3d.at[wid], hctx_v)
        pltpu.sync_copy(ctx3d.at[wid], ctx_v)
        pltpu.sync_copy(hcent3d.at[wid], hcid_v)
        pltpu.sync_copy(cent3d.at[wid], cid_v)

        def fire_gather(j, p):
            cpy = [pltpu.async_copy(
                in_embed2.at[hcid_v.at[j // 16, pl.ds((j % 16) * CB, CB)]],
                cent_bufs[p], gsems[p])]
            base = j * ROWS
            off = 0
            while off < ROWS:
                row, col = (base + off) // 128, (base + off) % 128
                n = min(128 - col, ROWS - off)
                cpy.append(pltpu.async_copy(
                    out_embed2.at[hctx_v.at[row, pl.ds(col, n)]],
                    rows_bufs[p].at[pl.ds(off, n)], gsems[p]))
                off += n
            return cpy

        gpend = {0: fire_gather(0, 0)}

        for j in range(n_chunks):
            p = j % 2
            for c in gpend.pop(j):
                c.wait()
            if j + 1 < n_chunks:
                gpend[j + 1] = fire_gather(j + 1, (j + 1) % 2)

            cent_j = cent_bufs[p]
            rows_j = rows_bufs[p]

            def group_body(g, _):
                lanes = lax.iota(jnp.int32, 16)
                prow = g * 16 + lanes
                k20 = jnp.full((16,), K, jnp.int32)
                brow = lax.div(prow, k20)
                base = j * ROWS + g * 16
                cx16 = ctx_v[base >> 7, pl.ds(base & 127, 16)]
                upar = (cx16 & 1) << 6
                bvec = j * CB + brow
                cw16 = plsc.load_gather(cid_v, [bvec >> 7, bvec & 127])
                vpar = (cw16 & 1) << 6

                def d_body(d, t):
                    dv = jnp.full((16,), d, jnp.int32)
                    u = plsc.load_gather(rows_j, [prow, upar + dv])
                    v = plsc.load_gather(cent_j, [brow, vpar + dv])
                    return t + u * v

                t = lax.fori_loop(0, DIM, d_body,
                                  jnp.zeros((16,), jnp.float32), unroll=4)
                out_v[pl.ds(base, 16)] = t
                return 0

            lax.fori_loop(0, groups, group_body, 0)

        pltpu.sync_copy(out_v, out_hbm.at[pl.ds(wid * pairs_per_w,
                                                pairs_per_w)])

    return k


def kernel(center_words, context_words, in_embed, out_embed):
    B, Kk = context_words.shape
    assert Kk == K and in_embed.shape[1] == DIM
    V = in_embed.shape[0]
    in2 = in_embed.reshape(V // 2, 2 * DIM)
    out2 = out_embed.reshape(V // 2, 2 * DIM)
    cw = center_words.astype(jnp.int32)
    cx = context_words.astype(jnp.int32)
    hctx3d = (cx >> 1).reshape(NW, B * K // NW // 128, 128)
    ctx3d = cx.reshape(NW, B * K // NW // 128, 128)
    cent4 = cw.reshape(NW, 4, 128)
    pad = jnp.zeros((NW, 4, 128), jnp.int32)
    cent3d = jnp.concatenate([cent4, pad], axis=1)
    hcent3d = jnp.concatenate([cent4 >> 1, pad], axis=1)
    out_flat = _sc_word2vec(B)(hctx3d, ctx3d, hcent3d, cent3d, in2, out2)
    return out_flat.reshape(B, K)
